# baseline (speedup 1.0000x reference)
import dataclasses
import functools

import jax
import jax.numpy as jnp
from jax import lax
from jax.experimental import pallas as pl
from jax.experimental.pallas import tpu as pltpu
from jax.experimental.pallas import tpu_sc as plsc

_N = 10000
_E = 320000
_ET = _E + _N
_EPAD = 331776
_NP = 10240
_B = 1000
_GRID = _N // _B


_CH = 48


def _make_edge_kernel(epc: int, hk: int):
    ch = _CH
    rw = hk * 32
    ow = rw + 16
    nch = epc // (16 * ch)
    nb = nch // 4
    rpt = _NP // 16
    mesh = plsc.VectorSubcoreMesh(core_axis_name="c", subcore_axis_name="s")
    cp = pltpu.CompilerParams()
    for fld, val in (("needs_layout_passes", False),
                     ("use_tc_tiling_on_sc", False)):
        if fld in pltpu.CompilerParams.__dataclass_fields__:
            cp = dataclasses.replace(cp, **{fld: val})

    @functools.partial(
        pl.kernel,
        out_type=jax.ShapeDtypeStruct((2 * _NP, ow), jnp.float32),
        mesh=mesh,
        compiler_params=cp,
        scratch_types=[
            pltpu.VMEM((ch, rw), jnp.float32),
            pltpu.VMEM((ch, rw), jnp.float32),
            pltpu.VMEM((ch, rw), jnp.float32),
            pltpu.VMEM((ch, rw), jnp.float32),
            pltpu.VMEM((ch, ow), jnp.float32),
            pltpu.VMEM((2, 3, ch), jnp.int32),
            pltpu.VMEM((2, 3, ch), jnp.int32),
            pltpu.VMEM((128,), jnp.float32),
            pltpu.VMEM((128, 16), jnp.float32),
            pltpu.VMEM_SHARED((_NP, ow), jnp.float32),
            pltpu.SemaphoreType.DMA,
            pltpu.SemaphoreType.DMA,
            pltpu.SemaphoreType.DMA,
            pltpu.SemaphoreType.DMA,
        ],
    )
    def edge_kernel(xl_hbm, xr_hbm, att_hbm, idx_hbm, out_hbm,
                    xlr0, xlr1, xrr0, xrr1, msg, sidx0, sidx1, attv, attb,
                    accum, sg0, sg1, si0, si1):
        c = lax.axis_index("c")
        s = lax.axis_index("s")
        xlr, xrr = [xlr0, xlr1], [xrr0, xrr1]
        sidx, sg, si = [sidx0, sidx1], [sg0, sg1], [si0, si1]
        pairbase = s * (nch // 2)

        @pl.loop(0, ch)
        def _(i):
            @pl.loop(0, ow, step=16)
            def _(j):
                msg[i, pl.ds(j, 16)] = jnp.zeros((16,), jnp.float32)

        for i in range(0, rpt, ch):
            pltpu.sync_copy(msg.at[pl.ds(0, min(ch, rpt - i))],
                            accum.at[pl.ds(s * rpt + i, min(ch, rpt - i))])
        pltpu.sync_copy(att_hbm.at[c], attv)
        lane = lax.iota(jnp.int32, 16)

        @pl.loop(0, rw)
        def _(i):
            attb[i, pl.ds(0, 16)] = plsc.load_gather(
                attv, [jnp.full((16,), 0, jnp.int32) + i])
        plsc.subcore_barrier()

        def issue_idx(q, pslot):
            pltpu.async_copy(idx_hbm.at[c, pairbase + pslot], sidx[q], si[q])

        def wait_idx(q, pslot):
            pltpu.make_async_copy(idx_hbm.at[c, pairbase + pslot],
                                  sidx[q], si[q]).wait()

        def issue_gather(q, r, p):
            pltpu.async_copy(xl_hbm.at[sidx[q].at[r, 0]], xlr[p], sg[p])
            pltpu.async_copy(xr_hbm.at[sidx[q].at[r, 1]], xrr[p], sg[p])

        def wait_gather(q, r, p):
            pltpu.make_async_copy(xl_hbm.at[sidx[q].at[r, 0]],
                                  xlr[p], sg[p]).wait()
            pltpu.make_async_copy(xr_hbm.at[sidx[q].at[r, 1]],
                                  xrr[p], sg[p]).wait()

        def compscat(p, q, r):
            xl_b, xr_b = xlr[p], xrr[p]

            @pl.loop(0, ch, step=16)
            def _(e0):
                rows = e0 + lane
                for h in range(hk):
                    o = h * 32
                    acc = [jnp.zeros((16,), jnp.float32) for _ in range(4)]
                    for cc in range(32):
                        cv = jnp.full((16,), o + cc, jnp.int32)
                        g1 = plsc.load_gather(xl_b, [rows, cv])
                        g2 = plsc.load_gather(xr_b, [rows, cv])
                        sv = g1 + g2
                        lv = jnp.maximum(sv, sv * 0.2)
                        acc[cc % 4] = acc[cc % 4] + lv * attb[o + cc,
                                                              pl.ds(0, 16)]
                    wv = jnp.exp((acc[0] + acc[1]) + (acc[2] + acc[3]))
                    plsc.store_scatter(
                        msg, [rows, jnp.full((16,), rw + h, jnp.int32)], wv)
                    for cc in range(32):
                        cv = jnp.full((16,), o + cc, jnp.int32)
                        g1 = plsc.load_gather(xl_b, [rows, cv])
                        plsc.store_scatter(msg, [rows, cv], g1 * wv)

            pltpu.sync_copy(msg, accum.at[sidx[q].at[r, 2]], add=True)

        pltpu.sync_copy(idx_hbm.at[c, pairbase], sidx[0])
        issue_gather(0, 0, 0)
        issue_idx(1, 1)

        @pl.loop(0, nb)
        def _(m):
            more = m < nb - 1
            wait_gather(0, 0, 0)
            issue_gather(0, 1, 1)
            compscat(0, 0, 0)

            wait_gather(0, 1, 1)
            wait_idx(1, 2 * m + 1)
            issue_gather(1, 0, 0)
            compscat(1, 0, 1)

            @pl.when(more)
            def _():
                issue_idx(0, 2 * m + 2)

            wait_gather(1, 0, 0)
            issue_gather(1, 1, 1)
            compscat(0, 1, 0)

            wait_gather(1, 1, 1)

            @pl.when(more)
            def _():
                wait_idx(0, 2 * m + 2)
                issue_gather(0, 0, 0)

            compscat(1, 1, 1)

            @pl.when(more)
            def _():
                issue_idx(1, 2 * m + 3)

        plsc.subcore_barrier()
        pltpu.sync_copy(accum.at[pl.ds(s * rpt, rpt)],
                        out_hbm.at[pl.ds(c * _NP + s * rpt, rpt)])

    return edge_kernel


_edge1 = _make_edge_kernel(_EPAD, 4)
_edge2 = _make_edge_kernel(_EPAD // 2, 1)


def _pre1_body(x_ref, wl, wr, wres, bres, xl_o, xr_o, r_o):
    xb = x_ref[...]
    xl_o[...] = jnp.dot(xb, wl[...], preferred_element_type=jnp.float32)
    xr_o[...] = jnp.dot(xb, wr[...], preferred_element_type=jnp.float32)
    r_o[...] = jnp.dot(xb, wres[...], preferred_element_type=jnp.float32) + bres[...]


def _layer_norm(v, g, b):
    mu = jnp.mean(v, axis=-1, keepdims=True)
    var = jnp.mean((v - mu) * (v - mu), axis=-1, keepdims=True)
    return (v - mu) / jnp.sqrt(var + 1e-5) * g + b


def _elu(v):
    return jnp.where(v > 0.0, v, jnp.exp(v) - 1.0)


def _mid_body(acc, r1_ref, b1_ref, g1_ref, be1_ref, wl2, wr2, wres2, bres2,
              xl2_o, xr2_o, r2_o):
    a = acc[...]
    parts = []
    for ci in range(2):
        for j in range(4):
            m = a[ci, :, 32 * j:32 * j + 32]
            d = a[ci, :, 128 + j][:, None]
            parts.append(m / (d + 1e-16))
    o1 = jnp.concatenate(parts, axis=-1) + b1_ref[...]
    h = _elu(_layer_norm(o1, g1_ref[...], be1_ref[...]) + r1_ref[...])
    xl2_o[...] = jnp.dot(h, wl2[...], preferred_element_type=jnp.float32)
    xr2_o[...] = jnp.dot(h, wr2[...], preferred_element_type=jnp.float32)
    r2_o[...] = jnp.dot(h, wres2[...], preferred_element_type=jnp.float32) + bres2[...]


def _post_body(acc, r2_ref, b2_ref, g2_ref, be2_ref, wout, bout, y_o):
    a = acc[...]
    m = a[0, :, 0:32] + a[1, :, 0:32]
    d = (a[0, :, 32] + a[1, :, 32])[:, None]
    o2 = m / (d + 1e-16) + b2_ref[...]
    h2 = _elu(_layer_norm(o2, g2_ref[...], be2_ref[...]) + r2_ref[...])
    y_o[...] = jnp.dot(h2, wout[...], preferred_element_type=jnp.float32) + bout[...]


def _full(shape):
    return pl.BlockSpec(shape, lambda i: tuple(0 for _ in shape))


def kernel(x, edge_index, Wl1, Wr1, att1, b1, g1, be1, Wres1, bres1,
           Wl2, Wr2, att2, b2, g2, be2, Wres2, bres2, Wout, bout):
    f32 = jnp.float32
    ar = jnp.arange(_N, dtype=jnp.int32)
    padi = jnp.full((_EPAD - _ET,), _N, jnp.int32)
    src = jnp.concatenate([edge_index[0].astype(jnp.int32), ar, padi])
    dst = jnp.concatenate([edge_index[1].astype(jnp.int32), ar, padi])

    def pack_idx(sg, dg, ds_):
        arr = jnp.stack([sg, dg, ds_], 0).reshape(3, -1, _CH)
        return arr.transpose(1, 0, 2).reshape(-1, 2, 3, _CH)

    idx1 = jnp.stack([pack_idx(src, dst, dst),
                      pack_idx(src + _NP, dst + _NP, dst)])
    sh, dh = src.reshape(2, -1), dst.reshape(2, -1)
    idx2 = jnp.stack([pack_idx(sh[i], dh[i], dh[i]) for i in range(2)])

    xl1, xr1, r1 = pl.pallas_call(
        _pre1_body,
        grid=(_GRID,),
        in_specs=[pl.BlockSpec((_B, 128), lambda i: (i, 0)),
                  _full((128, 256)), _full((128, 256)), _full((128, 256)),
                  _full((1, 256))],
        out_specs=[pl.BlockSpec((_B, 256), lambda i: (i, 0))] * 3,
        out_shape=[jax.ShapeDtypeStruct((_N, 256), f32)] * 3,
    )(x, Wl1, Wr1, Wres1, bres1.reshape(1, 256))

    def sc_table(v):
        t = v.reshape(_N, 2, 128).transpose(1, 0, 2)
        return jnp.pad(t, ((0, 0), (0, _NP - _N), (0, 0))).reshape(2 * _NP, 128)

    att1v = att1.reshape(2, 128)
    acc1 = _edge1(sc_table(xl1), sc_table(xr1), att1v, idx1)
    acc1 = acc1.reshape(2, _NP, 144)

    xl2, xr2, r2 = pl.pallas_call(
        _mid_body,
        grid=(_GRID,),
        in_specs=[pl.BlockSpec((2, _B, 144), lambda i: (0, i, 0)),
                  pl.BlockSpec((_B, 256), lambda i: (i, 0)),
                  _full((1, 256)), _full((1, 256)), _full((1, 256)),
                  _full((256, 32)), _full((256, 32)), _full((256, 32)),
                  _full((1, 32))],
        out_specs=[pl.BlockSpec((_B, 32), lambda i: (i, 0))] * 3,
        out_shape=[jax.ShapeDtypeStruct((_N, 32), f32)] * 3,
    )(acc1, r1, b1.reshape(1, 256), g1.reshape(1, 256), be1.reshape(1, 256),
      Wl2, Wr2, Wres2, bres2.reshape(1, 32))

    def sc_table2(v):
        return jnp.pad(v, ((0, _NP - _N), (0, 0)))

    att2v = jnp.zeros((2, 128), f32).at[:, :32].set(att2[0])
    acc2 = _edge2(sc_table2(xl2), sc_table2(xr2), att2v, idx2)
    acc2 = acc2.reshape(2, _NP, 48)

    y = pl.pallas_call(
        _post_body,
        grid=(_GRID,),
        in_specs=[pl.BlockSpec((2, _B, 48), lambda i: (0, i, 0)),
                  pl.BlockSpec((_B, 32), lambda i: (i, 0)),
                  _full((1, 32)), _full((1, 32)), _full((1, 32)),
                  _full((32, 64)), _full((1, 64))],
        out_specs=pl.BlockSpec((_B, 64), lambda i: (i, 0)),
        out_shape=jax.ShapeDtypeStruct((_N, 64), f32),
    )(acc2, r2, b2.reshape(1, 32), g2.reshape(1, 32), be2.reshape(1, 32),
      Wout, bout.reshape(1, 64))

    return y


# speedup vs baseline: 1.9220x; 1.9220x over previous
import dataclasses
import functools

import jax
import jax.numpy as jnp
from jax import lax
from jax.experimental import pallas as pl
from jax.experimental.pallas import tpu as pltpu
from jax.experimental.pallas import tpu_sc as plsc

_N = 10000
_E = 320000
_ET = _E + _N
_EPAD = 331776
_NP = 10240
_B = 1000
_GRID = _N // _B


_CH = 48


def _make_edge_kernel(epc: int, hk: int):
    ch = _CH
    rw = hk * 32
    ow = rw + 16
    nch = epc // (16 * ch)
    nb = nch // 4
    rpt = _NP // 16
    mesh = plsc.VectorSubcoreMesh(core_axis_name="c", subcore_axis_name="s")
    cp = pltpu.CompilerParams()
    for fld, val in (("needs_layout_passes", False),
                     ("use_tc_tiling_on_sc", False)):
        if fld in pltpu.CompilerParams.__dataclass_fields__:
            cp = dataclasses.replace(cp, **{fld: val})

    @functools.partial(
        pl.kernel,
        out_type=jax.ShapeDtypeStruct((2 * _NP, ow), jnp.float32),
        mesh=mesh,
        compiler_params=cp,
        scratch_types=[
            pltpu.VMEM((ch, rw), jnp.float32),
            pltpu.VMEM((ch, rw), jnp.float32),
            pltpu.VMEM((ch, rw), jnp.float32),
            pltpu.VMEM((ch, rw), jnp.float32),
            pltpu.VMEM((ch, ow), jnp.float32),
            pltpu.VMEM((2, 3, ch), jnp.int32),
            pltpu.VMEM((2, 3, ch), jnp.int32),
            pltpu.VMEM((128,), jnp.float32),
            pltpu.VMEM_SHARED((_NP, ow), jnp.float32),
            pltpu.SemaphoreType.DMA,
            pltpu.SemaphoreType.DMA,
            pltpu.SemaphoreType.DMA,
            pltpu.SemaphoreType.DMA,
        ],
    )
    def edge_kernel(xl_hbm, xr_hbm, att_hbm, idx_hbm, out_hbm,
                    xlr0, xlr1, xrr0, xrr1, msg, sidx0, sidx1, attv,
                    accum, sg0, sg1, si0, si1):
        c = lax.axis_index("c")
        s = lax.axis_index("s")
        xlr, xrr = [xlr0, xlr1], [xrr0, xrr1]
        sidx, sg, si = [sidx0, sidx1], [sg0, sg1], [si0, si1]
        pairbase = s * (nch // 2)

        @pl.loop(0, ch)
        def _(i):
            @pl.loop(0, ow, step=16)
            def _(j):
                msg[i, pl.ds(j, 16)] = jnp.zeros((16,), jnp.float32)

        for i in range(0, rpt, ch):
            pltpu.sync_copy(msg.at[pl.ds(0, min(ch, rpt - i))],
                            accum.at[pl.ds(s * rpt + i, min(ch, rpt - i))])
        pltpu.sync_copy(att_hbm.at[c], attv)
        plsc.subcore_barrier()

        lane = lax.iota(jnp.int32, 16)
        attc = [[attv[pl.ds(h * 32, 16)], attv[pl.ds(h * 32 + 16, 16)]]
                for h in range(hk)]
        masks = [lane == h for h in range(hk)]
        perms = [(lane ^ sh)[:, None] for sh in (8, 4, 2, 1)]
        dnums = lax.GatherDimensionNumbers(
            offset_dims=(), collapsed_slice_dims=(0,), start_index_map=(0,))

        def lane_perm(t, pm):
            return lax.gather(t, pm, dnums, slice_sizes=(1,),
                              mode=lax.GatherScatterMode.PROMISE_IN_BOUNDS)

        def issue_idx(q, pslot):
            pltpu.async_copy(idx_hbm.at[c, pairbase + pslot], sidx[q], si[q])

        def wait_idx(q, pslot):
            pltpu.make_async_copy(idx_hbm.at[c, pairbase + pslot],
                                  sidx[q], si[q]).wait()

        def issue_gather(q, r, p):
            pltpu.async_copy(xl_hbm.at[sidx[q].at[r, 0]], xlr[p], sg[p])
            pltpu.async_copy(xr_hbm.at[sidx[q].at[r, 1]], xrr[p], sg[p])

        def wait_gather(q, r, p):
            pltpu.make_async_copy(xl_hbm.at[sidx[q].at[r, 0]],
                                  xlr[p], sg[p]).wait()
            pltpu.make_async_copy(xr_hbm.at[sidx[q].at[r, 1]],
                                  xrr[p], sg[p]).wait()

        def compscat(p, q, r):
            xl_b, xr_b = xlr[p], xrr[p]

            @pl.loop(0, ch, step=4)
            def _(e0):
                for du in range(4):
                    e = e0 + du
                    wrow = jnp.zeros((16,), jnp.float32)
                    for h in range(hk):
                        o = h * 32
                        a0 = xl_b[e, pl.ds(o, 16)]
                        a1 = xl_b[e, pl.ds(o + 16, 16)]
                        s0 = a0 + xr_b[e, pl.ds(o, 16)]
                        s1 = a1 + xr_b[e, pl.ds(o + 16, 16)]
                        l0 = jnp.maximum(s0, s0 * 0.2)
                        l1 = jnp.maximum(s1, s1 * 0.2)
                        t = l0 * attc[h][0] + l1 * attc[h][1]
                        for pm in perms:
                            t = t + lane_perm(t, pm)
                        wv = jnp.exp(t)
                        msg[e, pl.ds(o, 16)] = a0 * wv
                        msg[e, pl.ds(o + 16, 16)] = a1 * wv
                        wrow = jnp.where(masks[h], wv, wrow)
                    msg[e, pl.ds(rw, 16)] = wrow

            pltpu.sync_copy(msg, accum.at[sidx[q].at[r, 2]], add=True)

        pltpu.sync_copy(idx_hbm.at[c, pairbase], sidx[0])
        issue_gather(0, 0, 0)
        issue_idx(1, 1)

        @pl.loop(0, nb)
        def _(m):
            more = m < nb - 1
            wait_gather(0, 0, 0)
            issue_gather(0, 1, 1)
            compscat(0, 0, 0)

            wait_gather(0, 1, 1)
            wait_idx(1, 2 * m + 1)
            issue_gather(1, 0, 0)
            compscat(1, 0, 1)

            @pl.when(more)
            def _():
                issue_idx(0, 2 * m + 2)

            wait_gather(1, 0, 0)
            issue_gather(1, 1, 1)
            compscat(0, 1, 0)

            wait_gather(1, 1, 1)

            @pl.when(more)
            def _():
                wait_idx(0, 2 * m + 2)
                issue_gather(0, 0, 0)

            compscat(1, 1, 1)

            @pl.when(more)
            def _():
                issue_idx(1, 2 * m + 3)

        plsc.subcore_barrier()
        pltpu.sync_copy(accum.at[pl.ds(s * rpt, rpt)],
                        out_hbm.at[pl.ds(c * _NP + s * rpt, rpt)])

    return edge_kernel


_edge1 = _make_edge_kernel(_EPAD, 4)
_edge2 = _make_edge_kernel(_EPAD // 2, 1)


def _pre1_body(x_ref, wl, wr, wres, bres, xl_o, xr_o, r_o):
    xb = x_ref[...]
    xl_o[...] = jnp.dot(xb, wl[...], preferred_element_type=jnp.float32)
    xr_o[...] = jnp.dot(xb, wr[...], preferred_element_type=jnp.float32)
    r_o[...] = jnp.dot(xb, wres[...], preferred_element_type=jnp.float32) + bres[...]


def _layer_norm(v, g, b):
    mu = jnp.mean(v, axis=-1, keepdims=True)
    var = jnp.mean((v - mu) * (v - mu), axis=-1, keepdims=True)
    return (v - mu) / jnp.sqrt(var + 1e-5) * g + b


def _elu(v):
    return jnp.where(v > 0.0, v, jnp.exp(v) - 1.0)


def _mid_body(acc, r1_ref, b1_ref, g1_ref, be1_ref, wl2, wr2, wres2, bres2,
              xl2_o, xr2_o, r2_o):
    a = acc[...]
    parts = []
    for ci in range(2):
        for j in range(4):
            m = a[ci, :, 32 * j:32 * j + 32]
            d = a[ci, :, 128 + j][:, None]
            parts.append(m / (d + 1e-16))
    o1 = jnp.concatenate(parts, axis=-1) + b1_ref[...]
    h = _elu(_layer_norm(o1, g1_ref[...], be1_ref[...]) + r1_ref[...])
    xl2_o[...] = jnp.dot(h, wl2[...], preferred_element_type=jnp.float32)
    xr2_o[...] = jnp.dot(h, wr2[...], preferred_element_type=jnp.float32)
    r2_o[...] = jnp.dot(h, wres2[...], preferred_element_type=jnp.float32) + bres2[...]


def _post_body(acc, r2_ref, b2_ref, g2_ref, be2_ref, wout, bout, y_o):
    a = acc[...]
    m = a[0, :, 0:32] + a[1, :, 0:32]
    d = (a[0, :, 32] + a[1, :, 32])[:, None]
    o2 = m / (d + 1e-16) + b2_ref[...]
    h2 = _elu(_layer_norm(o2, g2_ref[...], be2_ref[...]) + r2_ref[...])
    y_o[...] = jnp.dot(h2, wout[...], preferred_element_type=jnp.float32) + bout[...]


def _full(shape):
    return pl.BlockSpec(shape, lambda i: tuple(0 for _ in shape))


def kernel(x, edge_index, Wl1, Wr1, att1, b1, g1, be1, Wres1, bres1,
           Wl2, Wr2, att2, b2, g2, be2, Wres2, bres2, Wout, bout):
    f32 = jnp.float32
    ar = jnp.arange(_N, dtype=jnp.int32)
    padi = jnp.full((_EPAD - _ET,), _N, jnp.int32)
    src = jnp.concatenate([edge_index[0].astype(jnp.int32), ar, padi])
    dst = jnp.concatenate([edge_index[1].astype(jnp.int32), ar, padi])

    def pack_idx(sg, dg, ds_):
        arr = jnp.stack([sg, dg, ds_], 0).reshape(3, -1, _CH)
        return arr.transpose(1, 0, 2).reshape(-1, 2, 3, _CH)

    idx1 = jnp.stack([pack_idx(src, dst, dst),
                      pack_idx(src + _NP, dst + _NP, dst)])
    sh, dh = src.reshape(2, -1), dst.reshape(2, -1)
    idx2 = jnp.stack([pack_idx(sh[i], dh[i], dh[i]) for i in range(2)])

    xl1, xr1, r1 = pl.pallas_call(
        _pre1_body,
        grid=(_GRID,),
        in_specs=[pl.BlockSpec((_B, 128), lambda i: (i, 0)),
                  _full((128, 256)), _full((128, 256)), _full((128, 256)),
                  _full((1, 256))],
        out_specs=[pl.BlockSpec((_B, 256), lambda i: (i, 0))] * 3,
        out_shape=[jax.ShapeDtypeStruct((_N, 256), f32)] * 3,
    )(x, Wl1, Wr1, Wres1, bres1.reshape(1, 256))

    def sc_table(v):
        t = v.reshape(_N, 2, 128).transpose(1, 0, 2)
        return jnp.pad(t, ((0, 0), (0, _NP - _N), (0, 0))).reshape(2 * _NP, 128)

    att1v = att1.reshape(2, 128)
    acc1 = _edge1(sc_table(xl1), sc_table(xr1), att1v, idx1)
    acc1 = acc1.reshape(2, _NP, 144)

    xl2, xr2, r2 = pl.pallas_call(
        _mid_body,
        grid=(_GRID,),
        in_specs=[pl.BlockSpec((2, _B, 144), lambda i: (0, i, 0)),
                  pl.BlockSpec((_B, 256), lambda i: (i, 0)),
                  _full((1, 256)), _full((1, 256)), _full((1, 256)),
                  _full((256, 32)), _full((256, 32)), _full((256, 32)),
                  _full((1, 32))],
        out_specs=[pl.BlockSpec((_B, 32), lambda i: (i, 0))] * 3,
        out_shape=[jax.ShapeDtypeStruct((_N, 32), f32)] * 3,
    )(acc1, r1, b1.reshape(1, 256), g1.reshape(1, 256), be1.reshape(1, 256),
      Wl2, Wr2, Wres2, bres2.reshape(1, 32))

    def sc_table2(v):
        return jnp.pad(v, ((0, _NP - _N), (0, 0)))

    att2v = jnp.zeros((2, 128), f32).at[:, :32].set(att2[0])
    acc2 = _edge2(sc_table2(xl2), sc_table2(xr2), att2v, idx2)
    acc2 = acc2.reshape(2, _NP, 48)

    y = pl.pallas_call(
        _post_body,
        grid=(_GRID,),
        in_specs=[pl.BlockSpec((2, _B, 48), lambda i: (0, i, 0)),
                  pl.BlockSpec((_B, 32), lambda i: (i, 0)),
                  _full((1, 32)), _full((1, 32)), _full((1, 32)),
                  _full((32, 64)), _full((1, 64))],
        out_specs=pl.BlockSpec((_B, 64), lambda i: (i, 0)),
        out_shape=jax.ShapeDtypeStruct((_N, 64), f32),
    )(acc2, r2, b2.reshape(1, 32), g2.reshape(1, 32), be2.reshape(1, 32),
      Wout, bout.reshape(1, 64))

    return y


# speedup vs baseline: 1.9831x; 1.0318x over previous
import dataclasses
import functools

import jax
import jax.numpy as jnp
from jax import lax
from jax.experimental import pallas as pl
from jax.experimental.pallas import tpu as pltpu
from jax.experimental.pallas import tpu_sc as plsc

_N = 10000
_E = 320000
_ET = _E + _N
_EPAD = 331776
_NP = 10240
_B = 1000
_GRID = _N // _B


_CH = 32


def _make_edge_kernel(epc: int, hk: int):
    ch = _CH
    rw = hk * 32
    ow = rw + 16
    nch = epc // (16 * ch)
    nb = nch // 4
    rpt = _NP // 16
    mesh = plsc.VectorSubcoreMesh(core_axis_name="c", subcore_axis_name="s")
    cp = pltpu.CompilerParams()
    for fld, val in (("needs_layout_passes", False),
                     ("use_tc_tiling_on_sc", False)):
        if fld in pltpu.CompilerParams.__dataclass_fields__:
            cp = dataclasses.replace(cp, **{fld: val})

    @functools.partial(
        pl.kernel,
        out_type=jax.ShapeDtypeStruct((2 * _NP, ow), jnp.float32),
        mesh=mesh,
        compiler_params=cp,
        scratch_types=[
            pltpu.VMEM((ch, rw), jnp.float32),
            pltpu.VMEM((ch, rw), jnp.float32),
            pltpu.VMEM((ch, rw), jnp.float32),
            pltpu.VMEM((ch, rw), jnp.float32),
            pltpu.VMEM((ch, ow), jnp.float32),
            pltpu.VMEM((ch, ow), jnp.float32),
            pltpu.VMEM((2, 3, ch), jnp.int32),
            pltpu.VMEM((2, 3, ch), jnp.int32),
            pltpu.VMEM((ch,), jnp.int32),
            pltpu.VMEM((ch,), jnp.int32),
            pltpu.VMEM((128,), jnp.float32),
            pltpu.VMEM_SHARED((_NP, ow), jnp.float32),
            pltpu.SemaphoreType.DMA,
            pltpu.SemaphoreType.DMA,
            pltpu.SemaphoreType.DMA,
            pltpu.SemaphoreType.DMA,
            pltpu.SemaphoreType.DMA,
            pltpu.SemaphoreType.DMA,
        ],
    )
    def edge_kernel(xl_hbm, xr_hbm, att_hbm, idx_hbm, out_hbm,
                    xlr0, xlr1, xrr0, xrr1, msg0, msg1, sidx0, sidx1,
                    dstc0, dstc1, attv, accum, sg0, sg1, si0, si1, ss0, ss1):
        c = lax.axis_index("c")
        s = lax.axis_index("s")
        xlr, xrr = [xlr0, xlr1], [xrr0, xrr1]
        msgs, dstc = [msg0, msg1], [dstc0, dstc1]
        sidx, sg, si, ss = [sidx0, sidx1], [sg0, sg1], [si0, si1], [ss0, ss1]
        pairbase = s * (nch // 2)

        for msg in msgs:
            @pl.loop(0, ch)
            def _(i):
                @pl.loop(0, ow, step=16)
                def _(j):
                    msg[i, pl.ds(j, 16)] = jnp.zeros((16,), jnp.float32)

        for i in range(0, rpt, ch):
            pltpu.sync_copy(msgs[0].at[pl.ds(0, min(ch, rpt - i))],
                            accum.at[pl.ds(s * rpt + i, min(ch, rpt - i))])
        pltpu.sync_copy(att_hbm.at[c], attv)
        plsc.subcore_barrier()

        lane = lax.iota(jnp.int32, 16)
        attc = [[attv[pl.ds(h * 32, 16)], attv[pl.ds(h * 32 + 16, 16)]]
                for h in range(hk)]
        masks = [lane == h for h in range(hk)]
        perms = [(lane ^ sh)[:, None] for sh in (8, 4, 2, 1)]
        dnums = lax.GatherDimensionNumbers(
            offset_dims=(), collapsed_slice_dims=(0,), start_index_map=(0,))

        def lane_perm(t, pm):
            return lax.gather(t, pm, dnums, slice_sizes=(1,),
                              mode=lax.GatherScatterMode.PROMISE_IN_BOUNDS)

        def issue_idx(q, pslot):
            pltpu.async_copy(idx_hbm.at[c, pairbase + pslot], sidx[q], si[q])

        def wait_idx(q, pslot):
            pltpu.make_async_copy(idx_hbm.at[c, pairbase + pslot],
                                  sidx[q], si[q]).wait()

        def issue_gather(q, r, p):
            pltpu.async_copy(xl_hbm.at[sidx[q].at[r, 0]], xlr[p], sg[p])
            pltpu.async_copy(xr_hbm.at[sidx[q].at[r, 1]], xrr[p], sg[p])

        def wait_gather(q, r, p):
            pltpu.make_async_copy(xl_hbm.at[sidx[q].at[r, 0]],
                                  xlr[p], sg[p]).wait()
            pltpu.make_async_copy(xr_hbm.at[sidx[q].at[r, 1]],
                                  xrr[p], sg[p]).wait()

        def compute(p, q, r, mp):
            xl_b, xr_b, msg = xlr[p], xrr[p], msgs[mp]

            @pl.loop(0, ch, step=4)
            def _(e0):
                for du in range(4):
                    e = e0 + du
                    wrow = jnp.zeros((16,), jnp.float32)
                    for h in range(hk):
                        o = h * 32
                        a0 = xl_b[e, pl.ds(o, 16)]
                        a1 = xl_b[e, pl.ds(o + 16, 16)]
                        s0 = a0 + xr_b[e, pl.ds(o, 16)]
                        s1 = a1 + xr_b[e, pl.ds(o + 16, 16)]
                        l0 = jnp.maximum(s0, s0 * 0.2)
                        l1 = jnp.maximum(s1, s1 * 0.2)
                        t = l0 * attc[h][0] + l1 * attc[h][1]
                        for pm in perms:
                            t = t + lane_perm(t, pm)
                        wv = jnp.exp(t)
                        msg[e, pl.ds(o, 16)] = a0 * wv
                        msg[e, pl.ds(o + 16, 16)] = a1 * wv
                        wrow = jnp.where(masks[h], wv, wrow)
                    msg[e, pl.ds(rw, 16)] = wrow

            for i in range(0, ch, 16):
                dstc[mp][pl.ds(i, 16)] = sidx[q][r, 2, pl.ds(i, 16)]

        def issue_scat(mp):
            pltpu.async_copy(msgs[mp], accum.at[dstc[mp]], ss[mp], add=True)

        def wait_scat(mp):
            pltpu.make_async_copy(msgs[mp], accum.at[dstc[mp]],
                                  ss[mp]).wait()

        pltpu.sync_copy(idx_hbm.at[c, pairbase], sidx[0])
        issue_gather(0, 0, 0)
        issue_idx(1, 1)

        @pl.loop(0, nb)
        def _(m):
            more = m < nb - 1
            notfirst = m > 0

            wait_gather(0, 0, 0)
            issue_gather(0, 1, 1)

            @pl.when(notfirst)
            def _():
                wait_scat(0)

            compute(0, 0, 0, 0)
            issue_scat(0)

            wait_gather(0, 1, 1)
            wait_idx(1, 2 * m + 1)
            issue_gather(1, 0, 0)

            @pl.when(notfirst)
            def _():
                wait_scat(1)

            compute(1, 0, 1, 1)
            issue_scat(1)

            @pl.when(more)
            def _():
                issue_idx(0, 2 * m + 2)

            wait_gather(1, 0, 0)
            issue_gather(1, 1, 1)
            wait_scat(0)
            compute(0, 1, 0, 0)
            issue_scat(0)

            wait_gather(1, 1, 1)

            @pl.when(more)
            def _():
                wait_idx(0, 2 * m + 2)
                issue_gather(0, 0, 0)

            wait_scat(1)
            compute(1, 1, 1, 1)
            issue_scat(1)

            @pl.when(more)
            def _():
                issue_idx(1, 2 * m + 3)

        wait_scat(0)
        wait_scat(1)
        plsc.subcore_barrier()
        pltpu.sync_copy(accum.at[pl.ds(s * rpt, rpt)],
                        out_hbm.at[pl.ds(c * _NP + s * rpt, rpt)])

    return edge_kernel


_edge1 = _make_edge_kernel(_EPAD, 4)
_edge2 = _make_edge_kernel(_EPAD // 2, 1)


def _pre1_body(x_ref, wl, wr, wres, bres, xl_o, xr_o, r_o):
    xb = x_ref[...]
    xl_o[...] = jnp.dot(xb, wl[...], preferred_element_type=jnp.float32)
    xr_o[...] = jnp.dot(xb, wr[...], preferred_element_type=jnp.float32)
    r_o[...] = jnp.dot(xb, wres[...], preferred_element_type=jnp.float32) + bres[...]


def _layer_norm(v, g, b):
    mu = jnp.mean(v, axis=-1, keepdims=True)
    var = jnp.mean((v - mu) * (v - mu), axis=-1, keepdims=True)
    return (v - mu) / jnp.sqrt(var + 1e-5) * g + b


def _elu(v):
    return jnp.where(v > 0.0, v, jnp.exp(v) - 1.0)


def _mid_body(acc, r1_ref, b1_ref, g1_ref, be1_ref, wl2, wr2, wres2, bres2,
              xl2_o, xr2_o, r2_o):
    a = acc[...]
    parts = []
    for ci in range(2):
        for j in range(4):
            m = a[ci, :, 32 * j:32 * j + 32]
            d = a[ci, :, 128 + j][:, None]
            parts.append(m / (d + 1e-16))
    o1 = jnp.concatenate(parts, axis=-1) + b1_ref[...]
    h = _elu(_layer_norm(o1, g1_ref[...], be1_ref[...]) + r1_ref[...])
    xl2_o[...] = jnp.dot(h, wl2[...], preferred_element_type=jnp.float32)
    xr2_o[...] = jnp.dot(h, wr2[...], preferred_element_type=jnp.float32)
    r2_o[...] = jnp.dot(h, wres2[...], preferred_element_type=jnp.float32) + bres2[...]


def _post_body(acc, r2_ref, b2_ref, g2_ref, be2_ref, wout, bout, y_o):
    a = acc[...]
    m = a[0, :, 0:32] + a[1, :, 0:32]
    d = (a[0, :, 32] + a[1, :, 32])[:, None]
    o2 = m / (d + 1e-16) + b2_ref[...]
    h2 = _elu(_layer_norm(o2, g2_ref[...], be2_ref[...]) + r2_ref[...])
    y_o[...] = jnp.dot(h2, wout[...], preferred_element_type=jnp.float32) + bout[...]


def _full(shape):
    return pl.BlockSpec(shape, lambda i: tuple(0 for _ in shape))


def kernel(x, edge_index, Wl1, Wr1, att1, b1, g1, be1, Wres1, bres1,
           Wl2, Wr2, att2, b2, g2, be2, Wres2, bres2, Wout, bout):
    f32 = jnp.float32
    ar = jnp.arange(_N, dtype=jnp.int32)
    padi = jnp.full((_EPAD - _ET,), _N, jnp.int32)
    src = jnp.concatenate([edge_index[0].astype(jnp.int32), ar, padi])
    dst = jnp.concatenate([edge_index[1].astype(jnp.int32), ar, padi])

    def pack_idx(sg, dg, ds_):
        arr = jnp.stack([sg, dg, ds_], 0).reshape(3, -1, _CH)
        return arr.transpose(1, 0, 2).reshape(-1, 2, 3, _CH)

    idx1 = jnp.stack([pack_idx(src, dst, dst),
                      pack_idx(src + _NP, dst + _NP, dst)])
    sh, dh = src.reshape(2, -1), dst.reshape(2, -1)
    idx2 = jnp.stack([pack_idx(sh[i], dh[i], dh[i]) for i in range(2)])

    xl1, xr1, r1 = pl.pallas_call(
        _pre1_body,
        grid=(_GRID,),
        in_specs=[pl.BlockSpec((_B, 128), lambda i: (i, 0)),
                  _full((128, 256)), _full((128, 256)), _full((128, 256)),
                  _full((1, 256))],
        out_specs=[pl.BlockSpec((_B, 256), lambda i: (i, 0))] * 3,
        out_shape=[jax.ShapeDtypeStruct((_N, 256), f32)] * 3,
    )(x, Wl1, Wr1, Wres1, bres1.reshape(1, 256))

    def sc_table(v):
        t = v.reshape(_N, 2, 128).transpose(1, 0, 2)
        return jnp.pad(t, ((0, 0), (0, _NP - _N), (0, 0))).reshape(2 * _NP, 128)

    att1v = att1.reshape(2, 128)
    acc1 = _edge1(sc_table(xl1), sc_table(xr1), att1v, idx1)
    acc1 = acc1.reshape(2, _NP, 144)

    xl2, xr2, r2 = pl.pallas_call(
        _mid_body,
        grid=(_GRID,),
        in_specs=[pl.BlockSpec((2, _B, 144), lambda i: (0, i, 0)),
                  pl.BlockSpec((_B, 256), lambda i: (i, 0)),
                  _full((1, 256)), _full((1, 256)), _full((1, 256)),
                  _full((256, 32)), _full((256, 32)), _full((256, 32)),
                  _full((1, 32))],
        out_specs=[pl.BlockSpec((_B, 32), lambda i: (i, 0))] * 3,
        out_shape=[jax.ShapeDtypeStruct((_N, 32), f32)] * 3,
    )(acc1, r1, b1.reshape(1, 256), g1.reshape(1, 256), be1.reshape(1, 256),
      Wl2, Wr2, Wres2, bres2.reshape(1, 32))

    def sc_table2(v):
        return jnp.pad(v, ((0, _NP - _N), (0, 0)))

    att2v = jnp.zeros((2, 128), f32).at[:, :32].set(att2[0])
    acc2 = _edge2(sc_table2(xl2), sc_table2(xr2), att2v, idx2)
    acc2 = acc2.reshape(2, _NP, 48)

    y = pl.pallas_call(
        _post_body,
        grid=(_GRID,),
        in_specs=[pl.BlockSpec((2, _B, 48), lambda i: (0, i, 0)),
                  pl.BlockSpec((_B, 32), lambda i: (i, 0)),
                  _full((1, 32)), _full((1, 32)), _full((1, 32)),
                  _full((32, 64)), _full((1, 64))],
        out_specs=pl.BlockSpec((_B, 64), lambda i: (i, 0)),
        out_shape=jax.ShapeDtypeStruct((_N, 64), f32),
    )(acc2, r2, b2.reshape(1, 32), g2.reshape(1, 32), be2.reshape(1, 32),
      Wout, bout.reshape(1, 64))

    return y


# speedup vs baseline: 4.1847x; 2.1102x over previous
import dataclasses
import functools

import jax
import jax.numpy as jnp
from jax import lax
from jax.experimental import pallas as pl
from jax.experimental.pallas import tpu as pltpu
from jax.experimental.pallas import tpu_sc as plsc

_N = 10000
_E = 320000
_ET = _E + _N
_EPAD = 331776
_NP = 10240
_B = 1000
_GRID = _N // _B


_CH = 32


def _make_edge_kernel(epc: int, hk: int):
    ch = _CH
    rw = hk * 32
    ow = rw + 16
    nch = epc // (16 * ch)
    nb = nch // 4
    rpt = _NP // 16
    mesh = plsc.VectorSubcoreMesh(core_axis_name="c", subcore_axis_name="s")
    cp = pltpu.CompilerParams()
    for fld, val in (("needs_layout_passes", False),
                     ("use_tc_tiling_on_sc", False)):
        if fld in pltpu.CompilerParams.__dataclass_fields__:
            cp = dataclasses.replace(cp, **{fld: val})

    @functools.partial(
        pl.kernel,
        out_type=jax.ShapeDtypeStruct((2 * _NP, ow), jnp.float32),
        mesh=mesh,
        compiler_params=cp,
        scratch_types=[
            pltpu.VMEM((ch, rw), jnp.float32),
            pltpu.VMEM((ch, rw), jnp.float32),
            pltpu.VMEM((ch, rw), jnp.float32),
            pltpu.VMEM((ch, rw), jnp.float32),
            pltpu.VMEM((ch, ow), jnp.float32),
            pltpu.VMEM((ch, ow), jnp.float32),
            pltpu.VMEM((2, 3, ch), jnp.int32),
            pltpu.VMEM((2, 3, ch), jnp.int32),
            pltpu.VMEM((ch,), jnp.int32),
            pltpu.VMEM((ch,), jnp.int32),
            pltpu.VMEM((128,), jnp.float32),
            pltpu.VMEM_SHARED((_NP, ow), jnp.float32),
            pltpu.SemaphoreType.DMA,
            pltpu.SemaphoreType.DMA,
            pltpu.SemaphoreType.DMA,
            pltpu.SemaphoreType.DMA,
            pltpu.SemaphoreType.DMA,
            pltpu.SemaphoreType.DMA,
        ],
    )
    def edge_kernel(xl_hbm, xr_hbm, att_hbm, idx_hbm, out_hbm,
                    xlr0, xlr1, xrr0, xrr1, msg0, msg1, sidx0, sidx1,
                    dstc0, dstc1, attv, accum, sg0, sg1, si0, si1, ss0, ss1):
        c = lax.axis_index("c")
        s = lax.axis_index("s")
        xlr, xrr = [xlr0, xlr1], [xrr0, xrr1]
        msgs, dstc = [msg0, msg1], [dstc0, dstc1]
        sidx, sg, si, ss = [sidx0, sidx1], [sg0, sg1], [si0, si1], [ss0, ss1]
        pairbase = s * (nch // 2)

        for msg in msgs:
            @pl.loop(0, ch)
            def _(i):
                @pl.loop(0, ow, step=16)
                def _(j):
                    msg[i, pl.ds(j, 16)] = jnp.zeros((16,), jnp.float32)

        for i in range(0, rpt, ch):
            pltpu.sync_copy(msgs[0].at[pl.ds(0, min(ch, rpt - i))],
                            accum.at[pl.ds(s * rpt + i, min(ch, rpt - i))])
        pltpu.sync_copy(att_hbm.at[c], attv)
        plsc.subcore_barrier()

        lane = lax.iota(jnp.int32, 16)
        attc = [[attv[pl.ds(h * 32, 16)], attv[pl.ds(h * 32 + 16, 16)]]
                for h in range(hk)]
        masks = [lane == h for h in range(hk)]
        perms = [(lane ^ sh)[:, None] for sh in (8, 4, 2, 1)]
        dnums = lax.GatherDimensionNumbers(
            offset_dims=(), collapsed_slice_dims=(0,), start_index_map=(0,))

        def lane_perm(t, pm):
            return lax.gather(t, pm, dnums, slice_sizes=(1,),
                              mode=lax.GatherScatterMode.PROMISE_IN_BOUNDS)

        def issue_idx(q, pslot):
            pltpu.async_copy(idx_hbm.at[c, pairbase + pslot], sidx[q], si[q])

        def wait_idx(q, pslot):
            pltpu.make_async_copy(idx_hbm.at[c, pairbase + pslot],
                                  sidx[q], si[q]).wait()

        def issue_gather(q, r, p):
            pltpu.async_copy(xl_hbm.at[sidx[q].at[r, 0]], xlr[p], sg[p])
            pltpu.async_copy(xr_hbm.at[sidx[q].at[r, 1]], xrr[p], sg[p])

        def wait_gather(q, r, p):
            pltpu.make_async_copy(xl_hbm.at[sidx[q].at[r, 0]],
                                  xlr[p], sg[p]).wait()
            pltpu.make_async_copy(xr_hbm.at[sidx[q].at[r, 1]],
                                  xrr[p], sg[p]).wait()

        def compute(p, q, r, mp):
            xl_b, xr_b, msg = xlr[p], xrr[p], msgs[mp]

            @plsc.parallel_loop(0, ch, 1, unroll=4)
            def _(e):
                wrow = jnp.zeros((16,), jnp.float32)
                for h in range(hk):
                    o = h * 32
                    a0 = xl_b[e, pl.ds(o, 16)]
                    a1 = xl_b[e, pl.ds(o + 16, 16)]
                    s0 = a0 + xr_b[e, pl.ds(o, 16)]
                    s1 = a1 + xr_b[e, pl.ds(o + 16, 16)]
                    l0 = jnp.maximum(s0, s0 * 0.2)
                    l1 = jnp.maximum(s1, s1 * 0.2)
                    t = l0 * attc[h][0] + l1 * attc[h][1]
                    for pm in perms:
                        t = t + lane_perm(t, pm)
                    wv = jnp.exp(t)
                    msg[e, pl.ds(o, 16)] = a0 * wv
                    msg[e, pl.ds(o + 16, 16)] = a1 * wv
                    wrow = jnp.where(masks[h], wv, wrow)
                msg[e, pl.ds(rw, 16)] = wrow

            for i in range(0, ch, 16):
                dstc[mp][pl.ds(i, 16)] = sidx[q][r, 2, pl.ds(i, 16)]

        def issue_scat(mp):
            pltpu.async_copy(msgs[mp], accum.at[dstc[mp]], ss[mp], add=True)

        def wait_scat(mp):
            pltpu.make_async_copy(msgs[mp], accum.at[dstc[mp]],
                                  ss[mp]).wait()

        pltpu.sync_copy(idx_hbm.at[c, pairbase], sidx[0])
        issue_gather(0, 0, 0)
        issue_idx(1, 1)

        @pl.loop(0, nb)
        def _(m):
            more = m < nb - 1
            notfirst = m > 0

            wait_gather(0, 0, 0)
            issue_gather(0, 1, 1)

            @pl.when(notfirst)
            def _():
                wait_scat(0)

            compute(0, 0, 0, 0)
            issue_scat(0)

            wait_gather(0, 1, 1)
            wait_idx(1, 2 * m + 1)
            issue_gather(1, 0, 0)

            @pl.when(notfirst)
            def _():
                wait_scat(1)

            compute(1, 0, 1, 1)
            issue_scat(1)

            @pl.when(more)
            def _():
                issue_idx(0, 2 * m + 2)

            wait_gather(1, 0, 0)
            issue_gather(1, 1, 1)
            wait_scat(0)
            compute(0, 1, 0, 0)
            issue_scat(0)

            wait_gather(1, 1, 1)

            @pl.when(more)
            def _():
                wait_idx(0, 2 * m + 2)
                issue_gather(0, 0, 0)

            wait_scat(1)
            compute(1, 1, 1, 1)
            issue_scat(1)

            @pl.when(more)
            def _():
                issue_idx(1, 2 * m + 3)

        wait_scat(0)
        wait_scat(1)
        plsc.subcore_barrier()
        pltpu.sync_copy(accum.at[pl.ds(s * rpt, rpt)],
                        out_hbm.at[pl.ds(c * _NP + s * rpt, rpt)])

    return edge_kernel


_edge1 = _make_edge_kernel(_EPAD, 4)
_edge2 = _make_edge_kernel(_EPAD // 2, 1)


def _pre1_body(x_ref, wl, wr, wres, bres, xl_o, xr_o, r_o):
    xb = x_ref[...]
    xl_o[...] = jnp.dot(xb, wl[...], preferred_element_type=jnp.float32)
    xr_o[...] = jnp.dot(xb, wr[...], preferred_element_type=jnp.float32)
    r_o[...] = jnp.dot(xb, wres[...], preferred_element_type=jnp.float32) + bres[...]


def _layer_norm(v, g, b):
    mu = jnp.mean(v, axis=-1, keepdims=True)
    var = jnp.mean((v - mu) * (v - mu), axis=-1, keepdims=True)
    return (v - mu) / jnp.sqrt(var + 1e-5) * g + b


def _elu(v):
    return jnp.where(v > 0.0, v, jnp.exp(v) - 1.0)


def _mid_body(acc, r1_ref, b1_ref, g1_ref, be1_ref, wl2, wr2, wres2, bres2,
              xl2_o, xr2_o, r2_o):
    a = acc[...]
    parts = []
    for ci in range(2):
        for j in range(4):
            m = a[ci, :, 32 * j:32 * j + 32]
            d = a[ci, :, 128 + j][:, None]
            parts.append(m / (d + 1e-16))
    o1 = jnp.concatenate(parts, axis=-1) + b1_ref[...]
    h = _elu(_layer_norm(o1, g1_ref[...], be1_ref[...]) + r1_ref[...])
    xl2_o[...] = jnp.dot(h, wl2[...], preferred_element_type=jnp.float32)
    xr2_o[...] = jnp.dot(h, wr2[...], preferred_element_type=jnp.float32)
    r2_o[...] = jnp.dot(h, wres2[...], preferred_element_type=jnp.float32) + bres2[...]


def _post_body(acc, r2_ref, b2_ref, g2_ref, be2_ref, wout, bout, y_o):
    a = acc[...]
    m = a[0, :, 0:32] + a[1, :, 0:32]
    d = (a[0, :, 32] + a[1, :, 32])[:, None]
    o2 = m / (d + 1e-16) + b2_ref[...]
    h2 = _elu(_layer_norm(o2, g2_ref[...], be2_ref[...]) + r2_ref[...])
    y_o[...] = jnp.dot(h2, wout[...], preferred_element_type=jnp.float32) + bout[...]


def _full(shape):
    return pl.BlockSpec(shape, lambda i: tuple(0 for _ in shape))


def kernel(x, edge_index, Wl1, Wr1, att1, b1, g1, be1, Wres1, bres1,
           Wl2, Wr2, att2, b2, g2, be2, Wres2, bres2, Wout, bout):
    f32 = jnp.float32
    ar = jnp.arange(_N, dtype=jnp.int32)
    padi = jnp.full((_EPAD - _ET,), _N, jnp.int32)
    src = jnp.concatenate([edge_index[0].astype(jnp.int32), ar, padi])
    dst = jnp.concatenate([edge_index[1].astype(jnp.int32), ar, padi])

    def pack_idx(sg, dg, ds_):
        arr = jnp.stack([sg, dg, ds_], 0).reshape(3, -1, _CH)
        return arr.transpose(1, 0, 2).reshape(-1, 2, 3, _CH)

    idx1 = jnp.stack([pack_idx(src, dst, dst),
                      pack_idx(src + _NP, dst + _NP, dst)])
    sh, dh = src.reshape(2, -1), dst.reshape(2, -1)
    idx2 = jnp.stack([pack_idx(sh[i], dh[i], dh[i]) for i in range(2)])

    xl1, xr1, r1 = pl.pallas_call(
        _pre1_body,
        grid=(_GRID,),
        in_specs=[pl.BlockSpec((_B, 128), lambda i: (i, 0)),
                  _full((128, 256)), _full((128, 256)), _full((128, 256)),
                  _full((1, 256))],
        out_specs=[pl.BlockSpec((_B, 256), lambda i: (i, 0))] * 3,
        out_shape=[jax.ShapeDtypeStruct((_N, 256), f32)] * 3,
    )(x, Wl1, Wr1, Wres1, bres1.reshape(1, 256))

    def sc_table(v):
        t = v.reshape(_N, 2, 128).transpose(1, 0, 2)
        return jnp.pad(t, ((0, 0), (0, _NP - _N), (0, 0))).reshape(2 * _NP, 128)

    att1v = att1.reshape(2, 128)
    acc1 = _edge1(sc_table(xl1), sc_table(xr1), att1v, idx1)
    acc1 = acc1.reshape(2, _NP, 144)

    xl2, xr2, r2 = pl.pallas_call(
        _mid_body,
        grid=(_GRID,),
        in_specs=[pl.BlockSpec((2, _B, 144), lambda i: (0, i, 0)),
                  pl.BlockSpec((_B, 256), lambda i: (i, 0)),
                  _full((1, 256)), _full((1, 256)), _full((1, 256)),
                  _full((256, 32)), _full((256, 32)), _full((256, 32)),
                  _full((1, 32))],
        out_specs=[pl.BlockSpec((_B, 32), lambda i: (i, 0))] * 3,
        out_shape=[jax.ShapeDtypeStruct((_N, 32), f32)] * 3,
    )(acc1, r1, b1.reshape(1, 256), g1.reshape(1, 256), be1.reshape(1, 256),
      Wl2, Wr2, Wres2, bres2.reshape(1, 32))

    def sc_table2(v):
        return jnp.pad(v, ((0, _NP - _N), (0, 0)))

    att2v = jnp.zeros((2, 128), f32).at[:, :32].set(att2[0])
    acc2 = _edge2(sc_table2(xl2), sc_table2(xr2), att2v, idx2)
    acc2 = acc2.reshape(2, _NP, 48)

    y = pl.pallas_call(
        _post_body,
        grid=(_GRID,),
        in_specs=[pl.BlockSpec((2, _B, 48), lambda i: (0, i, 0)),
                  pl.BlockSpec((_B, 32), lambda i: (i, 0)),
                  _full((1, 32)), _full((1, 32)), _full((1, 32)),
                  _full((32, 64)), _full((1, 64))],
        out_specs=pl.BlockSpec((_B, 64), lambda i: (i, 0)),
        out_shape=jax.ShapeDtypeStruct((_N, 64), f32),
    )(acc2, r2, b2.reshape(1, 32), g2.reshape(1, 32), be2.reshape(1, 32),
      Wout, bout.reshape(1, 64))

    return y
